# baseline (speedup 1.0000x reference)
import numpy as np
import jax
import jax.numpy as jnp
from jax.experimental import pallas as pl
from jax.experimental.pallas import tpu as pltpu
from jax.experimental.pallas import tpu_sc as plsc

SIZE = 32000
PAD = 0
SMOOTH = 0.1
CONF = 1.0 - SMOOTH
EPS = SMOOTH / (SIZE - 2)
C_ROW = float(SMOOTH * np.log(EPS) + CONF * np.log(CONF))
N = 4096
D = 512
BC = 1280
NBLK = SIZE // BC

RT = 2048
RS = N - RT
NQ = 4
RQ = RT // NQ
NW = 32
RPW = RS // NW
NSL = SIZE // 16


def _tc_body(x0, x1, x2, x3, t0, t1, t2, t3, mu_ref, lv_ref, beta_ref,
             rec_ref, klb_ref):
    j = pl.program_id(0)
    partial = jnp.float32(0.0)
    lanes = jax.lax.broadcasted_iota(jnp.int32, (RQ, BC), 1)
    for xq, tq in ((x0, t0), (x1, t1), (x2, t2), (x3, t3)):
        t = tq[...]
        nonpad = (t != PAD).astype(jnp.float32)
        w_hit = (-CONF) * nonpad
        w_miss = (-EPS) * nonpad
        w = jnp.where(lanes == t - j * BC, w_hit, w_miss)
        partial = partial + jnp.sum(xq[...] * w)

    @pl.when(j == 0)
    def _():
        cnt = jnp.float32(0.0)
        x0c = jnp.float32(0.0)
        for xq, tq in ((x0, t0), (x1, t1), (x2, t2), (x3, t3)):
            nonpad = (tq[...] != PAD).astype(jnp.float32)
            cnt = cnt + jnp.sum(nonpad)
            x0c = x0c + jnp.sum(xq[:, 0:1] * nonpad)
        rec_ref[0, 0] = cnt * C_ROW + EPS * x0c
        lv = lv_ref[...]
        s = jnp.sum(1.0 + lv - mu_ref[...] * mu_ref[...] - jnp.exp(lv))
        klb_ref[0, 0] = (-0.5 / (N * D)) * s * beta_ref[0]

    rec_ref[0, 0] += partial


def _tc_call(x, t2, mu, logvar, beta):
    return pl.pallas_call(
        _tc_body,
        grid=(NBLK,),
        in_specs=[
            pl.BlockSpec((RQ, BC), lambda j: (0, j)),
            pl.BlockSpec((RQ, BC), lambda j: (1, j)),
            pl.BlockSpec((RQ, BC), lambda j: (2, j)),
            pl.BlockSpec((RQ, BC), lambda j: (3, j)),
            pl.BlockSpec((RQ, 1), lambda j: (0, 0)),
            pl.BlockSpec((RQ, 1), lambda j: (1, 0)),
            pl.BlockSpec((RQ, 1), lambda j: (2, 0)),
            pl.BlockSpec((RQ, 1), lambda j: (3, 0)),
            pl.BlockSpec((N, D), lambda j: (0, 0)),
            pl.BlockSpec((N, D), lambda j: (0, 0)),
            pl.BlockSpec(memory_space=pltpu.SMEM),
        ],
        out_specs=[
            pl.BlockSpec(memory_space=pltpu.SMEM),
            pl.BlockSpec(memory_space=pltpu.SMEM),
        ],
        out_shape=[
            jax.ShapeDtypeStruct((1, 1), jnp.float32),
            jax.ShapeDtypeStruct((1, 1), jnp.float32),
        ],
        compiler_params=pltpu.CompilerParams(
            vmem_limit_bytes=100 * 1024 * 1024,
        ),
    )(x, x, x, x, t2, t2, t2, t2, mu, logvar, beta)


def _sc_body(x_hbm, t_hbm, out_hbm, tloc, buf_a, buf_b, acc, sem_a, sem_b):
    c = jax.lax.axis_index("c")
    s = jax.lax.axis_index("s")
    wid = s * 2 + c
    base = RT + wid * RPW

    pltpu.sync_copy(t_hbm.at[pl.ds(base, RPW)], tloc.at[pl.ds(0, RPW)])
    acc[...] = jnp.zeros((16,), jnp.float32)
    pltpu.async_copy(x_hbm.at[base], buf_a.at[pl.ds(0, SIZE)], sem_a)
    pltpu.async_copy(x_hbm.at[base + 1], buf_b.at[pl.ds(0, SIZE)], sem_b)

    iota16 = jax.lax.iota(jnp.int32, 16)
    crow = jnp.where(iota16 == 0, C_ROW, 0.0).astype(jnp.float32)
    zf = jnp.zeros((16,), jnp.float32)

    @pl.loop(0, RPW // 2)
    def _outer(g):
        for b, (buf, sem) in enumerate(((buf_a, sem_a), (buf_b, sem_b))):
            r = g * 2 + b
            pltpu.make_async_copy(
                x_hbm.at[base + r], buf.at[pl.ds(0, SIZE)], sem).wait()
            t = tloc[pl.ds(r, 16)][0]

            def _inner(k, carry):
                a0, a1, a2, a3 = carry
                o = k * 256
                for u in range(16):
                    v = buf[pl.ds(o + u * 16, 16)]
                    if u % 4 == 0:
                        a0 = a0 + v
                    elif u % 4 == 1:
                        a1 = a1 + v
                    elif u % 4 == 2:
                        a2 = a2 + v
                    else:
                        a3 = a3 + v
                return a0, a1, a2, a3

            a0, a1, a2, a3 = jax.lax.fori_loop(
                0, NSL // 16, _inner, (zf, zf, zf, zf))
            total = (a0 + a1) + (a2 + a3)
            hit = buf[pl.ds(t, 16)]
            head = buf[pl.ds(0, 16)]
            corr = jnp.where(iota16 == 0,
                             hit * (EPS - CONF) + head * EPS, 0.0)
            contrib = (-EPS) * total + corr + crow
            npf = jnp.where(t != PAD, jnp.float32(1.0), jnp.float32(0.0))
            acc[...] = acc[...] + contrib * npf

            @pl.when(r + 2 < RPW)
            def _():
                pltpu.async_copy(
                    x_hbm.at[base + r + 2], buf.at[pl.ds(0, SIZE)], sem)

    pltpu.sync_copy(acc, out_hbm.at[wid])


def _sc_call(x, t_i32):
    return pl.kernel(
        _sc_body,
        out_type=jax.ShapeDtypeStruct((NW, 16), jnp.float32),
        mesh=plsc.VectorSubcoreMesh(core_axis_name="c", subcore_axis_name="s"),
        scratch_types=[
            pltpu.VMEM((RPW + 16,), jnp.int32),
            pltpu.VMEM((SIZE + 16,), jnp.float32),
            pltpu.VMEM((SIZE + 16,), jnp.float32),
            pltpu.VMEM((16,), jnp.float32),
            pltpu.SemaphoreType.DMA,
            pltpu.SemaphoreType.DMA,
        ],
    )(x, t_i32)


def kernel(x, target, mu, logvar, beta):
    t_i32 = target.astype(jnp.int32)
    t2 = t_i32.reshape(N, 1)
    rec, klb = _tc_call(x, t2, mu, logvar, beta)
    sc_part = _sc_call(x, t_i32)
    rec_loss = (rec[0, 0] + jnp.sum(sc_part)) / N
    return rec_loss, klb.reshape(1)


# speedup vs baseline: 1.0160x; 1.0160x over previous
import numpy as np
import jax
import jax.numpy as jnp
from jax.experimental import pallas as pl
from jax.experimental.pallas import tpu as pltpu
from jax.experimental.pallas import tpu_sc as plsc

SIZE = 32000
PAD = 0
SMOOTH = 0.1
CONF = 1.0 - SMOOTH
EPS = SMOOTH / (SIZE - 2)
C_ROW = float(SMOOTH * np.log(EPS) + CONF * np.log(CONF))
N = 4096
D = 512
BC = 1280
NBLK = SIZE // BC

RT = 2816
RS = N - RT
NQ = 4
RQ = RT // NQ
NW = 32
RPW = RS // NW
NSL = SIZE // 16


def _tc_body(x0, x1, x2, x3, t0, t1, t2, t3, mu_ref, lv_ref, beta_ref,
             rec_ref, klb_ref):
    j = pl.program_id(0)
    partial = jnp.float32(0.0)
    lanes = jax.lax.broadcasted_iota(jnp.int32, (RQ, BC), 1)
    for xq, tq in ((x0, t0), (x1, t1), (x2, t2), (x3, t3)):
        t = tq[...]
        nonpad = (t != PAD).astype(jnp.float32)
        w_hit = (-CONF) * nonpad
        w_miss = (-EPS) * nonpad
        w = jnp.where(lanes == t - j * BC, w_hit, w_miss)
        partial = partial + jnp.sum(xq[...] * w)

    @pl.when(j == 0)
    def _():
        cnt = jnp.float32(0.0)
        x0c = jnp.float32(0.0)
        for xq, tq in ((x0, t0), (x1, t1), (x2, t2), (x3, t3)):
            nonpad = (tq[...] != PAD).astype(jnp.float32)
            cnt = cnt + jnp.sum(nonpad)
            x0c = x0c + jnp.sum(xq[:, 0:1] * nonpad)
        rec_ref[0, 0] = cnt * C_ROW + EPS * x0c
        lv = lv_ref[...]
        s = jnp.sum(1.0 + lv - mu_ref[...] * mu_ref[...] - jnp.exp(lv))
        klb_ref[0, 0] = (-0.5 / (N * D)) * s * beta_ref[0]

    rec_ref[0, 0] += partial


def _tc_call(x, t2, mu, logvar, beta):
    return pl.pallas_call(
        _tc_body,
        grid=(NBLK,),
        in_specs=[
            pl.BlockSpec((RQ, BC), lambda j: (0, j)),
            pl.BlockSpec((RQ, BC), lambda j: (1, j)),
            pl.BlockSpec((RQ, BC), lambda j: (2, j)),
            pl.BlockSpec((RQ, BC), lambda j: (3, j)),
            pl.BlockSpec((RQ, 1), lambda j: (0, 0)),
            pl.BlockSpec((RQ, 1), lambda j: (1, 0)),
            pl.BlockSpec((RQ, 1), lambda j: (2, 0)),
            pl.BlockSpec((RQ, 1), lambda j: (3, 0)),
            pl.BlockSpec((N, D), lambda j: (0, 0)),
            pl.BlockSpec((N, D), lambda j: (0, 0)),
            pl.BlockSpec(memory_space=pltpu.SMEM),
        ],
        out_specs=[
            pl.BlockSpec(memory_space=pltpu.SMEM),
            pl.BlockSpec(memory_space=pltpu.SMEM),
        ],
        out_shape=[
            jax.ShapeDtypeStruct((1, 1), jnp.float32),
            jax.ShapeDtypeStruct((1, 1), jnp.float32),
        ],
        compiler_params=pltpu.CompilerParams(
            vmem_limit_bytes=100 * 1024 * 1024,
        ),
    )(x, x, x, x, t2, t2, t2, t2, mu, logvar, beta)


def _sc_body(x_hbm, t_hbm, out_hbm, tloc, buf_a, buf_b, acc, sem_a, sem_b):
    c = jax.lax.axis_index("c")
    s = jax.lax.axis_index("s")
    wid = s * 2 + c
    base = RT + wid * RPW

    pltpu.sync_copy(t_hbm.at[pl.ds(base, RPW)], tloc.at[pl.ds(0, RPW)])
    acc[...] = jnp.zeros((16,), jnp.float32)
    pltpu.async_copy(x_hbm.at[base], buf_a.at[pl.ds(0, SIZE)], sem_a)
    pltpu.async_copy(x_hbm.at[base + 1], buf_b.at[pl.ds(0, SIZE)], sem_b)

    iota16 = jax.lax.iota(jnp.int32, 16)
    crow = jnp.where(iota16 == 0, C_ROW, 0.0).astype(jnp.float32)
    zf = jnp.zeros((16,), jnp.float32)

    @pl.loop(0, RPW // 2)
    def _outer(g):
        for b, (buf, sem) in enumerate(((buf_a, sem_a), (buf_b, sem_b))):
            r = g * 2 + b
            pltpu.make_async_copy(
                x_hbm.at[base + r], buf.at[pl.ds(0, SIZE)], sem).wait()
            t = tloc[pl.ds(r, 16)][0]

            def _inner(k, carry):
                a0, a1, a2, a3 = carry
                o = k * 256
                for u in range(16):
                    v = buf[pl.ds(o + u * 16, 16)]
                    if u % 4 == 0:
                        a0 = a0 + v
                    elif u % 4 == 1:
                        a1 = a1 + v
                    elif u % 4 == 2:
                        a2 = a2 + v
                    else:
                        a3 = a3 + v
                return a0, a1, a2, a3

            a0, a1, a2, a3 = jax.lax.fori_loop(
                0, NSL // 16, _inner, (zf, zf, zf, zf), unroll=2)
            total = (a0 + a1) + (a2 + a3)
            hit = buf[pl.ds(t, 16)]
            head = buf[pl.ds(0, 16)]
            corr = jnp.where(iota16 == 0,
                             hit * (EPS - CONF) + head * EPS, 0.0)
            contrib = (-EPS) * total + corr + crow
            npf = jnp.where(t != PAD, jnp.float32(1.0), jnp.float32(0.0))
            acc[...] = acc[...] + contrib * npf

            @pl.when(r + 2 < RPW)
            def _():
                pltpu.async_copy(
                    x_hbm.at[base + r + 2], buf.at[pl.ds(0, SIZE)], sem)

    pltpu.sync_copy(acc, out_hbm.at[wid])


def _sc_call(x, t_i32):
    return pl.kernel(
        _sc_body,
        out_type=jax.ShapeDtypeStruct((NW, 16), jnp.float32),
        mesh=plsc.VectorSubcoreMesh(core_axis_name="c", subcore_axis_name="s"),
        scratch_types=[
            pltpu.VMEM((RPW + 16,), jnp.int32),
            pltpu.VMEM((SIZE + 16,), jnp.float32),
            pltpu.VMEM((SIZE + 16,), jnp.float32),
            pltpu.VMEM((16,), jnp.float32),
            pltpu.SemaphoreType.DMA,
            pltpu.SemaphoreType.DMA,
        ],
    )(x, t_i32)


def kernel(x, target, mu, logvar, beta):
    t_i32 = target.astype(jnp.int32)
    t2 = t_i32.reshape(N, 1)
    rec, klb = _tc_call(x, t2, mu, logvar, beta)
    sc_part = _sc_call(x, t_i32)
    rec_loss = (rec[0, 0] + jnp.sum(sc_part)) / N
    return rec_loss, klb.reshape(1)


# speedup vs baseline: 1.1144x; 1.0968x over previous
import numpy as np
import jax
import jax.numpy as jnp
from jax.experimental import pallas as pl
from jax.experimental.pallas import tpu as pltpu

SIZE = 32000
PAD = 0
SMOOTH = 0.1
CONF = 1.0 - SMOOTH
EPS = SMOOTH / (SIZE - 2)
C_ROW = float(SMOOTH * np.log(EPS) + CONF * np.log(CONF))
N = 4096
D = 512
BC = 1280
NBLK = SIZE // BC
NQ = 4
RQ = N // NQ


def _body(x0, x1, x2, x3, t0, t1, t2, t3, mu_ref, lv_ref, beta_ref,
          rec_ref, klb_ref):
    j = pl.program_id(0)
    partial = jnp.float32(0.0)
    lanes = jax.lax.broadcasted_iota(jnp.int32, (RQ, BC), 1)
    for xq, tq in ((x0, t0), (x1, t1), (x2, t2), (x3, t3)):
        t = tq[...]
        nonpad = (t != PAD).astype(jnp.float32)
        w = jnp.where(lanes == t - j * BC, (-CONF) * nonpad, (-EPS) * nonpad)
        partial = partial + jnp.sum(xq[...] * w)

    @pl.when(j == 0)
    def _():
        cnt = jnp.float32(0.0)
        x0c = jnp.float32(0.0)
        for xq, tq in ((x0, t0), (x1, t1), (x2, t2), (x3, t3)):
            nonpad = (tq[...] != PAD).astype(jnp.float32)
            cnt = cnt + jnp.sum(nonpad)
            x0c = x0c + jnp.sum(xq[:, 0:1] * nonpad)
        rec_ref[0, 0] = cnt * C_ROW + EPS * x0c
        lv = lv_ref[...]
        s = jnp.sum(1.0 + lv - mu_ref[...] * mu_ref[...] - jnp.exp(lv))
        klb_ref[0, 0] = (-0.5 / (N * D)) * s * beta_ref[0]

    rec_ref[0, 0] += partial


def kernel(x, target, mu, logvar, beta):
    t2 = target.astype(jnp.int32).reshape(N, 1)
    rec, klb = pl.pallas_call(
        _body,
        grid=(NBLK,),
        in_specs=[
            pl.BlockSpec((RQ, BC), lambda j: (0, j)),
            pl.BlockSpec((RQ, BC), lambda j: (1, j)),
            pl.BlockSpec((RQ, BC), lambda j: (2, j)),
            pl.BlockSpec((RQ, BC), lambda j: (3, j)),
            pl.BlockSpec((RQ, 1), lambda j: (0, 0)),
            pl.BlockSpec((RQ, 1), lambda j: (1, 0)),
            pl.BlockSpec((RQ, 1), lambda j: (2, 0)),
            pl.BlockSpec((RQ, 1), lambda j: (3, 0)),
            pl.BlockSpec((N, D), lambda j: (0, 0)),
            pl.BlockSpec((N, D), lambda j: (0, 0)),
            pl.BlockSpec(memory_space=pltpu.SMEM),
        ],
        out_specs=[
            pl.BlockSpec(memory_space=pltpu.SMEM),
            pl.BlockSpec(memory_space=pltpu.SMEM),
        ],
        out_shape=[
            jax.ShapeDtypeStruct((1, 1), jnp.float32),
            jax.ShapeDtypeStruct((1, 1), jnp.float32),
        ],
        compiler_params=pltpu.CompilerParams(
            vmem_limit_bytes=100 * 1024 * 1024,
        ),
    )(x, x, x, x, t2, t2, t2, t2, mu, logvar, beta)
    return rec[0, 0] / N, klb.reshape(1)
